# baseline (speedup 1.0000x reference)
import functools

import jax
import jax.numpy as jnp
from jax import lax
from jax.experimental import pallas as pl
from jax.experimental.pallas import tpu as pltpu
from jax.experimental.pallas import tpu_sc as plsc

N0 = 1408
E = 90112
T = 16
EPT = E // T
Q = 704
NQ = EPT // Q
STAGE = EPT + Q
NPAD = 1536
DUMPMAX = NPAD - 1

LAYERS = [
    (1408, 704, 88, 1408),
    (704, 352, 48, 768),
    (352, 176, 24, 384),
    (176, 88, 16, 256),
]
RMAX = 96
NEG = -3.0e38


def _mm_body(x_ref, w_ref, o_ref):
    o_ref[...] = jnp.dot(x_ref[...], w_ref[...], preferred_element_type=jnp.float32)


def _matmul(x, w):
    return pl.pallas_call(
        _mm_body,
        out_shape=jax.ShapeDtypeStruct((x.shape[0], w.shape[1]), jnp.float32),
    )(x, w)


def _head_body(x_ref, w1_ref, b1_ref, w2_ref, b2_ref, w3_ref, b3_ref, o_ref):
    h = jnp.dot(x_ref[...], w1_ref[...], preferred_element_type=jnp.float32) + b1_ref[...]
    h = jnp.dot(h, w2_ref[...], preferred_element_type=jnp.float32) + b2_ref[...]
    o_ref[...] = jnp.dot(h, w3_ref[...], preferred_element_type=jnp.float32) + b3_ref[...]


def _head(xf, w1, b1, w2, b2, w3, b3):
    return pl.pallas_call(
        _head_body,
        out_shape=jax.ShapeDtypeStruct((1, 96), jnp.float32),
    )(xf, w1, b1, w2, b2, w3, b3)


def _rsqrt_newton(d):
    i = lax.bitcast_convert_type(d, jnp.int32)
    i = jnp.int32(0x5F3759DF) - (i >> 1)
    y = lax.bitcast_convert_type(i, jnp.float32)
    for _ in range(3):
        y = y * (1.5 - 0.5 * d * y * y)
    return y


def _sc_body(xw1, srcH, dstH, wsH, bpH,
             out,
             sX0, sX1, sXS, sAgg, sDeg, sDeg16, sDinv, sScore, sRank,
             v_msg, v_one, v_src1, v_dst2, v_stD, v_rt, v_xsl, v_al, v_dgl, v_iota,
             v_na, v_nb, v_z, v_zd, v_d, v_sc, v_sc2, v_rk, v_scall,
             v_w, v_bp, m_ref, sem):
    t = lax.axis_index("s")
    ZR = NPAD // T
    zrow = t * ZR

    def _zfill(r, c):
        v_z[r] = jnp.zeros((16,), jnp.float32)
        return c
    lax.fori_loop(0, RMAX, _zfill, 0)

    def _zdfill(g, c):
        v_zd[pl.ds(g * 16, 16)] = jnp.zeros((16,), jnp.float32)
        return c
    lax.fori_loop(0, RMAX // 16, _zdfill, 0)

    def _ofill(g, c):
        v_one[pl.ds(g * 16, 16)] = jnp.ones((16,), jnp.float32)
        return c
    lax.fori_loop(0, Q // 16, _ofill, 0)

    def _ifill(g, c):
        v_iota[pl.ds(g * 16, 16)] = g * 16 + lax.iota(jnp.int32, 16)
        return c
    lax.fori_loop(0, NPAD // 16, _ifill, 0)
    pltpu.sync_copy(wsH, v_w)
    pltpu.sync_copy(bpH, v_bp)
    pltpu.sync_copy(srcH.at[pl.ds(t * EPT, EPT)], v_src1.at[pl.ds(0, EPT)])
    for q in range(NQ):
        pltpu.sync_copy(dstH.at[pl.ds(t * EPT + q * Q, Q)], v_dst2.at[q])
    pltpu.sync_copy(v_zd, sDeg.at[pl.ds(zrow, ZR)])

    xcur, xnxt = sX0, sX1
    for l, (n_in, k, rpt, NS) in enumerate(LAYERS):
        first = l == 0
        last = l == len(LAYERS) - 1
        ngrp = (rpt + 15) // 16
        row0 = t * rpt
        lanes = lax.iota(jnp.int32, 16)
        GPR = Q // 16
        if first:
            m = EPT
            nq = NQ
        else:
            m = m_ref[0]
            nq = (m + (Q - 1)) // Q

        pltpu.sync_copy(v_z.at[pl.ds(0, ZR)], sAgg.at[pl.ds(zrow, ZR)])
        if first:
            pltpu.sync_copy(v_z.at[pl.ds(0, ZR)], sDeg16.at[pl.ds(zrow, ZR)])
            pltpu.sync_copy(xw1.at[pl.ds(t * rpt, rpt)], xcur.at[pl.ds(t * rpt, rpt)])
        else:
            pltpu.sync_copy(xcur.at[pl.ds(row0, rpt)], v_na.at[pl.ds(0, rpt)])

            def _xw_row(r, c):
                xr = v_na[r]
                acc = jnp.zeros((16,), jnp.float32)
                for j in range(16):
                    acc = acc + xr[j] * v_w[(l - 1) * 16 + j]
                v_nb[r] = acc
                return c
            lax.fori_loop(0, rpt, _xw_row, 0)
            pltpu.sync_copy(v_nb.at[pl.ds(0, rpt)], xcur.at[pl.ds(row0, rpt)])
        plsc.subcore_barrier()

        if first:
            def _zdg(r, c):
                v_dgl[r] = jnp.zeros((16,), jnp.float32)
                return c
            lax.fori_loop(0, NPAD, _zdg, 0)
            ones16 = jnp.ones((16,), jnp.float32)

            def _dacc_g(g, c):
                dr = g // GPR
                dc = (g % GPR) * 16
                dv = v_dst2[dr, pl.ds(dc, 16)]
                for ri in range(16):
                    d_i = dv[ri]
                    v_dgl[d_i] = v_dgl[d_i] + ones16
                return c
            lax.fori_loop(0, EPT // 16, _dacc_g, 0)
            pltpu.sync_copy(v_dgl, sDeg16.at[v_iota], add=True)
            plsc.subcore_barrier()

        pltpu.sync_copy(xcur.at[pl.ds(zrow, ZR)], v_na)
        if first:
            pltpu.sync_copy(sDeg16.at[pl.ds(zrow, ZR)], v_al.at[pl.ds(0, ZR)])

            def _dinv16_grp(g, c):
                base = g * 16
                dacc = jnp.zeros((16,), jnp.float32)
                for ri in range(16):
                    y = _rsqrt_newton(v_al[base + ri] + 1.0)
                    v_na[base + ri] = y * v_na[base + ri]
                    dacc = jnp.where(lanes == ri, y[0], dacc)
                v_d[pl.ds(base, 16)] = dacc
                return c
            lax.fori_loop(0, ZR // 16, _dinv16_grp, 0)
        else:
            pltpu.sync_copy(sDeg.at[pl.ds(zrow, ZR)], v_d)

            def _dinv_grp(g, c):
                base = g * 16
                y = _rsqrt_newton(v_d[pl.ds(base, 16)] + 1.0)
                v_d[pl.ds(base, 16)] = y
                for ri in range(16):
                    v_na[base + ri] = y[ri] * v_na[base + ri]
                return c
            lax.fori_loop(0, ZR // 16, _dinv_grp, 0)
        pltpu.sync_copy(v_d, sDinv.at[pl.ds(zrow, ZR)])
        pltpu.sync_copy(v_na, sXS.at[pl.ds(zrow, ZR)])
        plsc.subcore_barrier()

        if first:
            pltpu.sync_copy(sXS, v_xsl)

            def _zal(r, c):
                v_al[r] = jnp.zeros((16,), jnp.float32)
                return c
            lax.fori_loop(0, NPAD, _zal, 0)

            def _macc_g(g, c):
                off = g * 16
                sv = v_src1[pl.ds(off, 16)]
                dr = g // GPR
                dc = (g % GPR) * 16
                dv = v_dst2[dr, pl.ds(dc, 16)]
                for ri in range(16):
                    row = v_xsl[sv[ri]]
                    d_i = dv[ri]
                    v_al[d_i] = v_al[d_i] + row
                return c
            lax.fori_loop(0, EPT // 16, _macc_g, 0)
            pltpu.sync_copy(v_al, sAgg.at[v_iota], add=True)
        else:
            def _msg_q(q, c):
                pltpu.sync_copy(sXS.at[v_src1.at[pl.ds(q * Q, Q)]], v_msg)
                pltpu.sync_copy(v_msg, sAgg.at[v_dst2.at[q]], add=True)
                return c
            lax.fori_loop(0, nq, _msg_q, 0)
        plsc.subcore_barrier()

        pltpu.sync_copy(sAgg.at[pl.ds(row0, rpt)], v_na.at[pl.ds(0, rpt)])
        pltpu.sync_copy(sXS.at[pl.ds(row0, rpt)], v_nb.at[pl.ds(0, rpt)])
        pltpu.sync_copy(sDinv.at[pl.ds(row0, rpt)], v_d.at[pl.ds(0, rpt)])
        bvec = v_bp[l]
        pvec = v_bp[4 + l]
        lanes = lax.iota(jnp.int32, 16)

        def _out_grp(g, c):
            base = g * 16
            dv = v_d[pl.ds(base, 16)]
            acc = jnp.full((16,), NEG, jnp.float32)
            for ri in range(16):
                o = dv[ri] * (v_na[base + ri] + v_nb[base + ri]) + bvec
                o = jnp.maximum(o, 0.0)
                v_na[base + ri] = o
                s = o * pvec
                for sh in (8, 4, 2, 1):
                    s = s + s.at[lanes ^ sh].get(mode='promise_in_bounds')
                acc = jnp.where(lanes == ri, s, acc)
            rowg = row0 + base + lanes
            v_sc[pl.ds(base, 16)] = jnp.where(rowg < n_in, acc, NEG)
            return c
        lax.fori_loop(0, ngrp, _out_grp, 0)
        pltpu.sync_copy(v_sc.at[pl.ds(0, rpt)], sScore.at[pl.ds(row0, rpt)])
        plsc.subcore_barrier()

        pltpu.sync_copy(sScore.at[pl.ds(0, NS)], v_scall.at[pl.ds(0, NS)])
        for g in range(ngrp):
            ibase = row0 + g * 16
            siv = v_scall[pl.ds(ibase, 16)]
            iota_i = ibase + lax.iota(jnp.int32, 16)

            def _rank_jg(jg, cnt):
                jbase = jg * 16
                sjv = v_scall[pl.ds(jbase, 16)]
                for jj in range(16):
                    sj = sjv[jj]
                    gt = sj > siv
                    eq = (sj == siv) & (jbase + jj < iota_i)
                    cnt = cnt + jnp.where(gt | eq, 1, 0)
                return cnt
            cnt = lax.fori_loop(0, NS // 16, _rank_jg,
                                jnp.zeros((16,), jnp.int32))
            if g * 16 + 16 > rpt:
                cnt = jnp.where(lanes < (rpt - g * 16), cnt, DUMPMAX)
            v_rk[pl.ds(g * 16, 16)] = cnt
        for g in range(ngrp, RMAX // 16):
            v_rk[pl.ds(g * 16, 16)] = jnp.full((16,), DUMPMAX, jnp.int32)
        pltpu.sync_copy(v_rk.at[pl.ds(0, rpt)], sRank.at[pl.ds(row0, rpt)])

        for g in range(RMAX // 16):
            sv = v_sc[pl.ds(g * 16, 16)]
            ev = jnp.exp(2.0 * sv)
            v_sc2[pl.ds(g * 16, 16)] = 1.0 - 2.0 / (ev + 1.0)

        def _scale_grp(g, c):
            base = g * 16
            tv = v_sc2[pl.ds(base, 16)]
            for ri in range(16):
                v_nb[base + ri] = v_na[base + ri] * tv[ri]
            return c
        lax.fori_loop(0, ngrp, _scale_grp, 0)
        pltpu.sync_copy(v_nb, xnxt.at[v_rk])
        if not last:
            pltpu.sync_copy(v_zd, sDeg.at[pl.ds(zrow, ZR)])
        plsc.subcore_barrier()

        if not last:
            pltpu.sync_copy(sRank.at[pl.ds(0, NS)], v_rt.at[pl.ds(0, NS)])
            ngq = (m + 15) // 16
            GPR = Q // 16

            def _cmp_g(g, mm):
                off = g * 16
                sv = v_src1[pl.ds(off, 16)]
                dr = g // GPR
                dc = (g % GPR) * 16
                dv = v_dst2[dr, pl.ds(dc, 16)]
                rs = plsc.load_gather(v_rt, [sv])
                rd = plsc.load_gather(v_rt, [dv])
                live = (rs < k) & (rd < k) & (off + lanes < m)
                key = jnp.where(live, 0, 1).astype(jnp.uint32)
                _, rs2 = plsc.sort_key_val(key, rs)
                _, rd2 = plsc.sort_key_val(key, rd)
                v_src1[pl.ds(mm, 16)] = rs2
                v_stD[pl.ds(mm, 16)] = rd2
                pc = plsc.all_reduce_population_count(live)
                return mm + pc[0]
            mm = lax.fori_loop(0, ngq, _cmp_g, jnp.int32(0))
            for g in range(Q // 16):
                v_src1[pl.ds(mm + g * 16, 16)] = jnp.full((16,), k, jnp.int32)
                v_stD[pl.ds(mm + g * 16, 16)] = jnp.full((16,), k, jnp.int32)
            nq2 = (mm + (Q - 1)) // Q

            def _cp_q(q, c):
                for gg in range(Q // 16):
                    v_dst2[q, pl.ds(gg * 16, 16)] = v_stD[pl.ds(q * Q + gg * 16, 16)]
                pltpu.sync_copy(v_one, sDeg.at[v_dst2.at[q]], add=True)
                return c
            lax.fori_loop(0, nq2, _cp_q, 0)
            m_ref[0] = mm
            plsc.subcore_barrier()
        xcur, xnxt = xnxt, xcur

    @pl.when(t == 0)
    def _():
        pltpu.sync_copy(xcur.at[pl.ds(0, 88)], out)


_sc_forward = functools.partial(
    pl.kernel,
    out_type=jax.ShapeDtypeStruct((88, 16), jnp.float32),
    mesh=plsc.VectorSubcoreMesh(core_axis_name="c", subcore_axis_name="s",
                                num_cores=1),
    compiler_params=pltpu.CompilerParams(use_tc_tiling_on_sc=False,
                                         needs_layout_passes=False),
    scratch_types=[
        pltpu.VMEM_SHARED((NPAD, 16), jnp.float32),
        pltpu.VMEM_SHARED((NPAD, 16), jnp.float32),
        pltpu.VMEM_SHARED((NPAD, 16), jnp.float32),
        pltpu.VMEM_SHARED((NPAD, 16), jnp.float32),
        pltpu.VMEM_SHARED((NPAD,), jnp.float32),
        pltpu.VMEM_SHARED((NPAD, 16), jnp.float32),
        pltpu.VMEM_SHARED((NPAD,), jnp.float32),
        pltpu.VMEM_SHARED((NPAD,), jnp.float32),
        pltpu.VMEM_SHARED((NPAD,), jnp.int32),
        pltpu.VMEM((Q, 16), jnp.float32),
        pltpu.VMEM((Q,), jnp.float32),
        pltpu.VMEM((STAGE,), jnp.int32),
        pltpu.VMEM((NQ, Q), jnp.int32),
        pltpu.VMEM((STAGE,), jnp.int32),
        pltpu.VMEM((1424,), jnp.int32),
        pltpu.VMEM((NPAD, 16), jnp.float32),
        pltpu.VMEM((NPAD, 16), jnp.float32),
        pltpu.VMEM((NPAD, 16), jnp.float32),
        pltpu.VMEM((NPAD,), jnp.int32),
        pltpu.VMEM((RMAX, 16), jnp.float32),
        pltpu.VMEM((RMAX, 16), jnp.float32),
        pltpu.VMEM((RMAX, 16), jnp.float32),
        pltpu.VMEM((RMAX,), jnp.float32),
        pltpu.VMEM((RMAX,), jnp.float32),
        pltpu.VMEM((RMAX,), jnp.float32),
        pltpu.VMEM((RMAX,), jnp.float32),
        pltpu.VMEM((RMAX,), jnp.int32),
        pltpu.VMEM((NPAD,), jnp.float32),
        pltpu.VMEM((48, 16), jnp.float32),
        pltpu.VMEM((8, 16), jnp.float32),
        pltpu.SMEM((8,), jnp.int32),
        pltpu.SemaphoreType.DMA,
    ],
)(_sc_body)


def kernel(x, edge_index, batch, W1, b1, p1, W2, b2, p2, W3, b3, p3, W4, b4, p4,
           fc1_W, fc1_b, fc2_W, fc2_b, fc3_W, fc3_b):
    src = edge_index[0]
    dst = edge_index[1]
    xw1 = _matmul(x, W1)
    Wst = jnp.concatenate([W2, W3, W4], axis=0)
    bpst = jnp.stack([
        b1, b2, b3, b4,
        p1 / jnp.linalg.norm(p1), p2 / jnp.linalg.norm(p2),
        p3 / jnp.linalg.norm(p3), p4 / jnp.linalg.norm(p4),
    ])
    x4 = _sc_forward(xw1, src, dst, Wst, bpst)
    out = _head(x4.reshape(1, N0), fc1_W, fc1_b.reshape(1, -1),
                fc2_W, fc2_b.reshape(1, -1), fc3_W, fc3_b.reshape(1, -1))
    return out.reshape(-1)


# speedup vs baseline: 1.2718x; 1.2718x over previous
import functools

import jax
import jax.numpy as jnp
from jax import lax
from jax.experimental import pallas as pl
from jax.experimental.pallas import tpu as pltpu
from jax.experimental.pallas import tpu_sc as plsc

N0 = 1408
E = 90112
T = 16
EPT = E // T
Q = 704
NQ = EPT // Q
STAGE = EPT + Q
NPAD = 1536
DUMPMAX = NPAD - 1

LAYERS = [
    (1408, 704, 88, 1408),
    (704, 352, 48, 768),
    (352, 176, 24, 384),
    (176, 88, 16, 256),
]
RMAX = 96
NEG = -3.0e38


def _mm_body(x_ref, w_ref, o_ref):
    o_ref[...] = jnp.dot(x_ref[...], w_ref[...], preferred_element_type=jnp.float32)


def _matmul(x, w):
    return pl.pallas_call(
        _mm_body,
        out_shape=jax.ShapeDtypeStruct((x.shape[0], w.shape[1]), jnp.float32),
    )(x, w)


def _head_body(x_ref, w1_ref, b1_ref, w2_ref, b2_ref, w3_ref, b3_ref, o_ref):
    h = jnp.dot(x_ref[...], w1_ref[...], preferred_element_type=jnp.float32) + b1_ref[...]
    h = jnp.dot(h, w2_ref[...], preferred_element_type=jnp.float32) + b2_ref[...]
    o_ref[...] = jnp.dot(h, w3_ref[...], preferred_element_type=jnp.float32) + b3_ref[...]


def _head(xf, w1, b1, w2, b2, w3, b3):
    return pl.pallas_call(
        _head_body,
        out_shape=jax.ShapeDtypeStruct((1, 96), jnp.float32),
    )(xf, w1, b1, w2, b2, w3, b3)


def _rsqrt_newton(d):
    i = lax.bitcast_convert_type(d, jnp.int32)
    i = jnp.int32(0x5F3759DF) - (i >> 1)
    y = lax.bitcast_convert_type(i, jnp.float32)
    for _ in range(3):
        y = y * (1.5 - 0.5 * d * y * y)
    return y


def _sc_body(xw1, srcH, dstH, wsH, bpH,
             out,
             sX0, sX1, sXS, sAgg, sDeg, sDinv, sScore, sRank,
             v_msg, v_one, v_src1, v_dst2, v_stD, v_rt,
             v_na, v_nb, v_z, v_zd, v_d, v_sc, v_sc2, v_rk, v_scall,
             v_w, v_bp, m_ref, sem):
    t = lax.axis_index("s")
    ZR = NPAD // T
    zrow = t * ZR

    def _zfill(r, c):
        v_z[r] = jnp.zeros((16,), jnp.float32)
        return c
    lax.fori_loop(0, RMAX, _zfill, 0)

    def _zdfill(g, c):
        v_zd[pl.ds(g * 16, 16)] = jnp.zeros((16,), jnp.float32)
        return c
    lax.fori_loop(0, RMAX // 16, _zdfill, 0)

    def _ofill(g, c):
        v_one[pl.ds(g * 16, 16)] = jnp.ones((16,), jnp.float32)
        return c
    lax.fori_loop(0, Q // 16, _ofill, 0)
    pltpu.sync_copy(wsH, v_w)
    pltpu.sync_copy(bpH, v_bp)
    pltpu.sync_copy(srcH.at[pl.ds(t * EPT, EPT)], v_src1.at[pl.ds(0, EPT)])
    for q in range(NQ):
        pltpu.sync_copy(dstH.at[pl.ds(t * EPT + q * Q, Q)], v_dst2.at[q])
    pltpu.sync_copy(v_zd, sDeg.at[pl.ds(zrow, ZR)])

    xcur, xnxt = sX0, sX1
    for l, (n_in, k, rpt, NS) in enumerate(LAYERS):
        first = l == 0
        last = l == len(LAYERS) - 1
        ngrp = (rpt + 15) // 16
        row0 = t * rpt
        lanes = lax.iota(jnp.int32, 16)
        GPR = Q // 16
        if first:
            m = EPT
            nq = NQ
        else:
            m = m_ref[0]
            nq = (m + (Q - 1)) // Q

        pltpu.sync_copy(v_z.at[pl.ds(0, ZR)], sAgg.at[pl.ds(zrow, ZR)])
        if first:
            pltpu.sync_copy(xw1.at[pl.ds(t * rpt, rpt)], xcur.at[pl.ds(t * rpt, rpt)])
        else:
            pltpu.sync_copy(xcur.at[pl.ds(row0, rpt)], v_na.at[pl.ds(0, rpt)])

            def _xw_row(r, c):
                xr = v_na[r]
                acc = jnp.zeros((16,), jnp.float32)
                for j in range(16):
                    acc = acc + xr[j] * v_w[(l - 1) * 16 + j]
                v_nb[r] = acc
                return c
            lax.fori_loop(0, rpt, _xw_row, 0)
            pltpu.sync_copy(v_nb.at[pl.ds(0, rpt)], xcur.at[pl.ds(row0, rpt)])
        plsc.subcore_barrier()

        if first:
            for q in range(NQ):
                pltpu.sync_copy(v_one, sDeg.at[v_dst2.at[q]], add=True)
            plsc.subcore_barrier()

        pltpu.sync_copy(sDeg.at[pl.ds(zrow, ZR)], v_d)
        pltpu.sync_copy(xcur.at[pl.ds(zrow, ZR)], v_na)

        def _dinv_grp(g, c):
            base = g * 16
            y = _rsqrt_newton(v_d[pl.ds(base, 16)] + 1.0)
            v_d[pl.ds(base, 16)] = y
            for ri in range(16):
                v_na[base + ri] = y[ri] * v_na[base + ri]
            return c
        lax.fori_loop(0, ZR // 16, _dinv_grp, 0)
        pltpu.sync_copy(v_d, sDinv.at[pl.ds(zrow, ZR)])
        pltpu.sync_copy(v_na, sXS.at[pl.ds(zrow, ZR)])
        plsc.subcore_barrier()

        def _msg_q(q, c):
            pltpu.sync_copy(sXS.at[v_src1.at[pl.ds(q * Q, Q)]], v_msg)
            pltpu.sync_copy(v_msg, sAgg.at[v_dst2.at[q]], add=True)
            return c
        if first:
            for q in range(NQ):
                _msg_q(q, 0)
        else:
            lax.fori_loop(0, nq, _msg_q, 0)
        plsc.subcore_barrier()

        pltpu.sync_copy(sAgg.at[pl.ds(row0, rpt)], v_na.at[pl.ds(0, rpt)])
        pltpu.sync_copy(sXS.at[pl.ds(row0, rpt)], v_nb.at[pl.ds(0, rpt)])
        pltpu.sync_copy(sDinv.at[pl.ds(row0, rpt)], v_d.at[pl.ds(0, rpt)])
        bvec = v_bp[l]
        pvec = v_bp[4 + l]
        lanes = lax.iota(jnp.int32, 16)

        def _out_grp(g, c):
            base = g * 16
            dv = v_d[pl.ds(base, 16)]
            acc = jnp.full((16,), NEG, jnp.float32)
            for ri in range(16):
                o = dv[ri] * (v_na[base + ri] + v_nb[base + ri]) + bvec
                o = jnp.maximum(o, 0.0)
                v_na[base + ri] = o
                s = o * pvec
                for sh in (8, 4, 2, 1):
                    s = s + s.at[lanes ^ sh].get(mode='promise_in_bounds')
                acc = jnp.where(lanes == ri, s, acc)
            rowg = row0 + base + lanes
            v_sc[pl.ds(base, 16)] = jnp.where(rowg < n_in, acc, NEG)
            return c
        lax.fori_loop(0, ngrp, _out_grp, 0)
        pltpu.sync_copy(v_sc.at[pl.ds(0, rpt)], sScore.at[pl.ds(row0, rpt)])
        plsc.subcore_barrier()

        pltpu.sync_copy(sScore.at[pl.ds(0, NS)], v_scall.at[pl.ds(0, NS)])
        for g in range(ngrp):
            ibase = row0 + g * 16
            siv = v_scall[pl.ds(ibase, 16)]
            iota_i = ibase + lax.iota(jnp.int32, 16)

            def _rank_jg(jg, cnt):
                jbase = jg * 16
                sjv = v_scall[pl.ds(jbase, 16)]
                for jj in range(16):
                    sj = sjv[jj]
                    gt = sj > siv
                    eq = (sj == siv) & (jbase + jj < iota_i)
                    cnt = cnt + jnp.where(gt | eq, 1, 0)
                return cnt
            cnt = lax.fori_loop(0, NS // 16, _rank_jg,
                                jnp.zeros((16,), jnp.int32))
            if g * 16 + 16 > rpt:
                cnt = jnp.where(lanes < (rpt - g * 16), cnt, DUMPMAX)
            v_rk[pl.ds(g * 16, 16)] = cnt
        for g in range(ngrp, RMAX // 16):
            v_rk[pl.ds(g * 16, 16)] = jnp.full((16,), DUMPMAX, jnp.int32)
        pltpu.sync_copy(v_rk.at[pl.ds(0, rpt)], sRank.at[pl.ds(row0, rpt)])

        for g in range(RMAX // 16):
            sv = v_sc[pl.ds(g * 16, 16)]
            ev = jnp.exp(2.0 * sv)
            v_sc2[pl.ds(g * 16, 16)] = 1.0 - 2.0 / (ev + 1.0)

        def _scale_grp(g, c):
            base = g * 16
            tv = v_sc2[pl.ds(base, 16)]
            for ri in range(16):
                v_nb[base + ri] = v_na[base + ri] * tv[ri]
            return c
        lax.fori_loop(0, ngrp, _scale_grp, 0)
        pltpu.sync_copy(v_nb, xnxt.at[v_rk])
        if not last:
            pltpu.sync_copy(v_zd, sDeg.at[pl.ds(zrow, ZR)])
        plsc.subcore_barrier()

        if not last:
            pltpu.sync_copy(sRank.at[pl.ds(0, NS)], v_rt.at[pl.ds(0, NS)])
            ngq = (m + 15) // 16
            GPR = Q // 16

            def _cmp_g(g, mm):
                off = g * 16
                sv = v_src1[pl.ds(off, 16)]
                dr = g // GPR
                dc = (g % GPR) * 16
                dv = v_dst2[dr, pl.ds(dc, 16)]
                rs = plsc.load_gather(v_rt, [sv])
                rd = plsc.load_gather(v_rt, [dv])
                live = (rs < k) & (rd < k) & (off + lanes < m)
                key = jnp.where(live, 0, 1).astype(jnp.uint32)
                _, rs2 = plsc.sort_key_val(key, rs)
                _, rd2 = plsc.sort_key_val(key, rd)
                v_src1[pl.ds(mm, 16)] = rs2
                v_stD[pl.ds(mm, 16)] = rd2
                pc = plsc.all_reduce_population_count(live)
                return mm + pc[0]
            mm = lax.fori_loop(0, ngq, _cmp_g, jnp.int32(0))
            for g in range(Q // 16):
                v_src1[pl.ds(mm + g * 16, 16)] = jnp.full((16,), k, jnp.int32)
                v_stD[pl.ds(mm + g * 16, 16)] = jnp.full((16,), k, jnp.int32)
            nq2 = (mm + (Q - 1)) // Q

            def _cp_q(q, c):
                for gg in range(Q // 16):
                    v_dst2[q, pl.ds(gg * 16, 16)] = v_stD[pl.ds(q * Q + gg * 16, 16)]
                pltpu.sync_copy(v_one, sDeg.at[v_dst2.at[q]], add=True)
                return c
            lax.fori_loop(0, nq2, _cp_q, 0)
            m_ref[0] = mm
            plsc.subcore_barrier()
        xcur, xnxt = xnxt, xcur

    @pl.when(t == 0)
    def _():
        pltpu.sync_copy(xcur.at[pl.ds(0, 88)], out)


_sc_forward = functools.partial(
    pl.kernel,
    out_type=jax.ShapeDtypeStruct((88, 16), jnp.float32),
    mesh=plsc.VectorSubcoreMesh(core_axis_name="c", subcore_axis_name="s",
                                num_cores=1),
    compiler_params=pltpu.CompilerParams(use_tc_tiling_on_sc=False,
                                         needs_layout_passes=False),
    scratch_types=[
        pltpu.VMEM_SHARED((NPAD, 16), jnp.float32),
        pltpu.VMEM_SHARED((NPAD, 16), jnp.float32),
        pltpu.VMEM_SHARED((NPAD, 16), jnp.float32),
        pltpu.VMEM_SHARED((NPAD, 16), jnp.float32),
        pltpu.VMEM_SHARED((NPAD,), jnp.float32),
        pltpu.VMEM_SHARED((NPAD,), jnp.float32),
        pltpu.VMEM_SHARED((NPAD,), jnp.float32),
        pltpu.VMEM_SHARED((NPAD,), jnp.int32),
        pltpu.VMEM((Q, 16), jnp.float32),
        pltpu.VMEM((Q,), jnp.float32),
        pltpu.VMEM((STAGE,), jnp.int32),
        pltpu.VMEM((NQ, Q), jnp.int32),
        pltpu.VMEM((STAGE,), jnp.int32),
        pltpu.VMEM((1424,), jnp.int32),
        pltpu.VMEM((RMAX, 16), jnp.float32),
        pltpu.VMEM((RMAX, 16), jnp.float32),
        pltpu.VMEM((RMAX, 16), jnp.float32),
        pltpu.VMEM((RMAX,), jnp.float32),
        pltpu.VMEM((RMAX,), jnp.float32),
        pltpu.VMEM((RMAX,), jnp.float32),
        pltpu.VMEM((RMAX,), jnp.float32),
        pltpu.VMEM((RMAX,), jnp.int32),
        pltpu.VMEM((NPAD,), jnp.float32),
        pltpu.VMEM((48, 16), jnp.float32),
        pltpu.VMEM((8, 16), jnp.float32),
        pltpu.SMEM((8,), jnp.int32),
        pltpu.SemaphoreType.DMA,
    ],
)(_sc_body)


def kernel(x, edge_index, batch, W1, b1, p1, W2, b2, p2, W3, b3, p3, W4, b4, p4,
           fc1_W, fc1_b, fc2_W, fc2_b, fc3_W, fc3_b):
    src = edge_index[0]
    dst = edge_index[1]
    xw1 = _matmul(x, W1)
    Wst = jnp.concatenate([W2, W3, W4], axis=0)
    bpst = jnp.stack([
        b1, b2, b3, b4,
        p1 / jnp.linalg.norm(p1), p2 / jnp.linalg.norm(p2),
        p3 / jnp.linalg.norm(p3), p4 / jnp.linalg.norm(p4),
    ])
    x4 = _sc_forward(xw1, src, dst, Wst, bpst)
    out = _head(x4.reshape(1, N0), fc1_W, fc1_b.reshape(1, -1),
                fc2_W, fc2_b.reshape(1, -1), fc3_W, fc3_b.reshape(1, -1))
    return out.reshape(-1)


# speedup vs baseline: 1.2848x; 1.0102x over previous
import functools

import jax
import jax.numpy as jnp
from jax import lax
from jax.experimental import pallas as pl
from jax.experimental.pallas import tpu as pltpu
from jax.experimental.pallas import tpu_sc as plsc

N0 = 1408
E = 90112
T = 16
EPT = E // T
Q = 704
NQ = EPT // Q
STAGE = EPT + Q
NPAD = 1536
DUMPMAX = NPAD - 1

LAYERS = [
    (1408, 704, 88, 1408),
    (704, 352, 48, 768),
    (352, 176, 24, 384),
    (176, 88, 16, 256),
]
RMAX = 96
NEG = -3.0e38


def _mm_body(x_ref, w_ref, o_ref):
    o_ref[...] = jnp.dot(x_ref[...], w_ref[...], preferred_element_type=jnp.float32)


def _matmul(x, w):
    return pl.pallas_call(
        _mm_body,
        out_shape=jax.ShapeDtypeStruct((x.shape[0], w.shape[1]), jnp.float32),
    )(x, w)


def _head_body(x_ref, w1_ref, b1_ref, w2_ref, b2_ref, w3_ref, b3_ref, o_ref):
    h = jnp.dot(x_ref[...], w1_ref[...], preferred_element_type=jnp.float32) + b1_ref[...]
    h = jnp.dot(h, w2_ref[...], preferred_element_type=jnp.float32) + b2_ref[...]
    o_ref[...] = jnp.dot(h, w3_ref[...], preferred_element_type=jnp.float32) + b3_ref[...]


def _head(xf, w1, b1, w2, b2, w3, b3):
    return pl.pallas_call(
        _head_body,
        out_shape=jax.ShapeDtypeStruct((1, 96), jnp.float32),
    )(xf, w1, b1, w2, b2, w3, b3)


def _rsqrt_newton(d):
    i = lax.bitcast_convert_type(d, jnp.int32)
    i = jnp.int32(0x5F3759DF) - (i >> 1)
    y = lax.bitcast_convert_type(i, jnp.float32)
    for _ in range(3):
        y = y * (1.5 - 0.5 * d * y * y)
    return y


def _sc_body(xw1, srcH, dstH, wsH, bpH,
             out,
             sX0, sX1, sXS, sAgg, sDeg, sDinv, sScore, sRank,
             v_msg, v_msg2, v_one, v_src1, v_dst2, v_stD, v_rt,
             v_na, v_nb, v_z, v_zd, v_d, v_sc, v_sc2, v_rk, v_scall,
             v_w, v_bp, m_ref, sem, sem2):
    t = lax.axis_index("s")
    ZR = NPAD // T
    zrow = t * ZR

    def _zfill(r, c):
        v_z[r] = jnp.zeros((16,), jnp.float32)
        return c
    lax.fori_loop(0, RMAX, _zfill, 0)

    def _zdfill(g, c):
        v_zd[pl.ds(g * 16, 16)] = jnp.zeros((16,), jnp.float32)
        return c
    lax.fori_loop(0, RMAX // 16, _zdfill, 0)

    def _ofill(g, c):
        v_one[pl.ds(g * 16, 16)] = jnp.ones((16,), jnp.float32)
        return c
    lax.fori_loop(0, Q // 16, _ofill, 0)
    pltpu.sync_copy(wsH, v_w)
    pltpu.sync_copy(bpH, v_bp)
    pltpu.sync_copy(srcH.at[pl.ds(t * EPT, EPT)], v_src1.at[pl.ds(0, EPT)])
    for q in range(NQ):
        pltpu.sync_copy(dstH.at[pl.ds(t * EPT + q * Q, Q)], v_dst2.at[q])
    pltpu.sync_copy(v_zd, sDeg.at[pl.ds(zrow, ZR)])

    xcur, xnxt = sX0, sX1
    for l, (n_in, k, rpt, NS) in enumerate(LAYERS):
        first = l == 0
        last = l == len(LAYERS) - 1
        ngrp = (rpt + 15) // 16
        row0 = t * rpt
        lanes = lax.iota(jnp.int32, 16)
        GPR = Q // 16
        if first:
            m = EPT
            nq = NQ
        else:
            m = m_ref[0]
            nq = (m + (Q - 1)) // Q

        pltpu.sync_copy(v_z.at[pl.ds(0, ZR)], sAgg.at[pl.ds(zrow, ZR)])
        if first:
            pltpu.sync_copy(xw1.at[pl.ds(t * rpt, rpt)], xcur.at[pl.ds(t * rpt, rpt)])
        else:
            pltpu.sync_copy(xcur.at[pl.ds(row0, rpt)], v_na.at[pl.ds(0, rpt)])

            def _xw_row(r, c):
                xr = v_na[r]
                acc = jnp.zeros((16,), jnp.float32)
                for j in range(16):
                    acc = acc + xr[j] * v_w[(l - 1) * 16 + j]
                v_nb[r] = acc
                return c
            lax.fori_loop(0, rpt, _xw_row, 0)
            pltpu.sync_copy(v_nb.at[pl.ds(0, rpt)], xcur.at[pl.ds(row0, rpt)])
        plsc.subcore_barrier()

        if first:
            for q in range(NQ):
                pltpu.sync_copy(v_one, sDeg.at[v_dst2.at[q]], add=True)
            plsc.subcore_barrier()

        pltpu.sync_copy(sDeg.at[pl.ds(zrow, ZR)], v_d)
        pltpu.sync_copy(xcur.at[pl.ds(zrow, ZR)], v_na)

        def _dinv_grp(g, c):
            base = g * 16
            y = _rsqrt_newton(v_d[pl.ds(base, 16)] + 1.0)
            v_d[pl.ds(base, 16)] = y
            for ri in range(16):
                v_na[base + ri] = y[ri] * v_na[base + ri]
            return c
        lax.fori_loop(0, ZR // 16, _dinv_grp, 0)
        pltpu.sync_copy(v_d, sDinv.at[pl.ds(zrow, ZR)])
        pltpu.sync_copy(v_na, sXS.at[pl.ds(zrow, ZR)])
        plsc.subcore_barrier()

        def _msg_q(q, c):
            pltpu.sync_copy(sXS.at[v_src1.at[pl.ds(q * Q, Q)]], v_msg)
            pltpu.sync_copy(v_msg, sAgg.at[v_dst2.at[q]], add=True)
            return c
        if first:
            bufs = [v_msg, v_msg2]
            sems = [sem, sem2]
            desc = pltpu.async_copy(sXS.at[v_src1.at[pl.ds(0, Q)]], bufs[0], sems[0])
            for q in range(NQ):
                desc.wait()
                if q + 1 < NQ:
                    desc = pltpu.async_copy(
                        sXS.at[v_src1.at[pl.ds((q + 1) * Q, Q)]],
                        bufs[(q + 1) % 2], sems[(q + 1) % 2])
                pltpu.sync_copy(bufs[q % 2], sAgg.at[v_dst2.at[q]], add=True)
        else:
            lax.fori_loop(0, nq, _msg_q, 0)
        plsc.subcore_barrier()

        pltpu.sync_copy(sAgg.at[pl.ds(row0, rpt)], v_na.at[pl.ds(0, rpt)])
        pltpu.sync_copy(sXS.at[pl.ds(row0, rpt)], v_nb.at[pl.ds(0, rpt)])
        pltpu.sync_copy(sDinv.at[pl.ds(row0, rpt)], v_d.at[pl.ds(0, rpt)])
        bvec = v_bp[l]
        pvec = v_bp[4 + l]
        lanes = lax.iota(jnp.int32, 16)

        def _out_grp(g, c):
            base = g * 16
            dv = v_d[pl.ds(base, 16)]
            acc = jnp.full((16,), NEG, jnp.float32)
            for ri in range(16):
                o = dv[ri] * (v_na[base + ri] + v_nb[base + ri]) + bvec
                o = jnp.maximum(o, 0.0)
                v_na[base + ri] = o
                s = o * pvec
                for sh in (8, 4, 2, 1):
                    s = s + s.at[lanes ^ sh].get(mode='promise_in_bounds')
                acc = jnp.where(lanes == ri, s, acc)
            rowg = row0 + base + lanes
            v_sc[pl.ds(base, 16)] = jnp.where(rowg < n_in, acc, NEG)
            return c
        lax.fori_loop(0, ngrp, _out_grp, 0)
        pltpu.sync_copy(v_sc.at[pl.ds(0, rpt)], sScore.at[pl.ds(row0, rpt)])
        plsc.subcore_barrier()

        pltpu.sync_copy(sScore.at[pl.ds(0, NS)], v_scall.at[pl.ds(0, NS)])
        for g in range(ngrp):
            ibase = row0 + g * 16
            siv = v_scall[pl.ds(ibase, 16)]
            iota_i = ibase + lax.iota(jnp.int32, 16)

            def _rank_jg(jg, cnt):
                jbase = jg * 16
                sjv = v_scall[pl.ds(jbase, 16)]
                for jj in range(16):
                    sj = sjv[jj]
                    gt = sj > siv
                    eq = (sj == siv) & (jbase + jj < iota_i)
                    cnt = cnt + jnp.where(gt | eq, 1, 0)
                return cnt
            cnt = lax.fori_loop(0, NS // 16, _rank_jg,
                                jnp.zeros((16,), jnp.int32))
            if g * 16 + 16 > rpt:
                cnt = jnp.where(lanes < (rpt - g * 16), cnt, DUMPMAX)
            v_rk[pl.ds(g * 16, 16)] = cnt
        for g in range(ngrp, RMAX // 16):
            v_rk[pl.ds(g * 16, 16)] = jnp.full((16,), DUMPMAX, jnp.int32)
        pltpu.sync_copy(v_rk.at[pl.ds(0, rpt)], sRank.at[pl.ds(row0, rpt)])

        for g in range(RMAX // 16):
            sv = v_sc[pl.ds(g * 16, 16)]
            ev = jnp.exp(2.0 * sv)
            v_sc2[pl.ds(g * 16, 16)] = 1.0 - 2.0 / (ev + 1.0)

        def _scale_grp(g, c):
            base = g * 16
            tv = v_sc2[pl.ds(base, 16)]
            for ri in range(16):
                v_nb[base + ri] = v_na[base + ri] * tv[ri]
            return c
        lax.fori_loop(0, ngrp, _scale_grp, 0)
        pltpu.sync_copy(v_nb, xnxt.at[v_rk])
        if not last:
            pltpu.sync_copy(v_zd, sDeg.at[pl.ds(zrow, ZR)])
        plsc.subcore_barrier()

        if not last:
            pltpu.sync_copy(sRank.at[pl.ds(0, NS)], v_rt.at[pl.ds(0, NS)])
            ngq = (m + 15) // 16
            GPR = Q // 16

            def _cmp_g(g, mm):
                off = g * 16
                sv = v_src1[pl.ds(off, 16)]
                dr = g // GPR
                dc = (g % GPR) * 16
                dv = v_dst2[dr, pl.ds(dc, 16)]
                rs = plsc.load_gather(v_rt, [sv])
                rd = plsc.load_gather(v_rt, [dv])
                live = (rs < k) & (rd < k) & (off + lanes < m)
                key = jnp.where(live, 0, 1).astype(jnp.uint32)
                _, rs2 = plsc.sort_key_val(key, rs)
                _, rd2 = plsc.sort_key_val(key, rd)
                v_src1[pl.ds(mm, 16)] = rs2
                v_stD[pl.ds(mm, 16)] = rd2
                pc = plsc.all_reduce_population_count(live)
                return mm + pc[0]
            mm = lax.fori_loop(0, ngq, _cmp_g, jnp.int32(0))
            for g in range(Q // 16):
                v_src1[pl.ds(mm + g * 16, 16)] = jnp.full((16,), k, jnp.int32)
                v_stD[pl.ds(mm + g * 16, 16)] = jnp.full((16,), k, jnp.int32)
            nq2 = (mm + (Q - 1)) // Q

            def _cp_q(q, c):
                for gg in range(Q // 16):
                    v_dst2[q, pl.ds(gg * 16, 16)] = v_stD[pl.ds(q * Q + gg * 16, 16)]
                pltpu.sync_copy(v_one, sDeg.at[v_dst2.at[q]], add=True)
                return c
            lax.fori_loop(0, nq2, _cp_q, 0)
            m_ref[0] = mm
            plsc.subcore_barrier()
        xcur, xnxt = xnxt, xcur

    @pl.when(t == 0)
    def _():
        pltpu.sync_copy(xcur.at[pl.ds(0, 88)], out)


_sc_forward = functools.partial(
    pl.kernel,
    out_type=jax.ShapeDtypeStruct((88, 16), jnp.float32),
    mesh=plsc.VectorSubcoreMesh(core_axis_name="c", subcore_axis_name="s",
                                num_cores=1),
    compiler_params=pltpu.CompilerParams(use_tc_tiling_on_sc=False,
                                         needs_layout_passes=False),
    scratch_types=[
        pltpu.VMEM_SHARED((NPAD, 16), jnp.float32),
        pltpu.VMEM_SHARED((NPAD, 16), jnp.float32),
        pltpu.VMEM_SHARED((NPAD, 16), jnp.float32),
        pltpu.VMEM_SHARED((NPAD, 16), jnp.float32),
        pltpu.VMEM_SHARED((NPAD,), jnp.float32),
        pltpu.VMEM_SHARED((NPAD,), jnp.float32),
        pltpu.VMEM_SHARED((NPAD,), jnp.float32),
        pltpu.VMEM_SHARED((NPAD,), jnp.int32),
        pltpu.VMEM((Q, 16), jnp.float32),
        pltpu.VMEM((Q, 16), jnp.float32),
        pltpu.VMEM((Q,), jnp.float32),
        pltpu.VMEM((STAGE,), jnp.int32),
        pltpu.VMEM((NQ, Q), jnp.int32),
        pltpu.VMEM((STAGE,), jnp.int32),
        pltpu.VMEM((1424,), jnp.int32),
        pltpu.VMEM((RMAX, 16), jnp.float32),
        pltpu.VMEM((RMAX, 16), jnp.float32),
        pltpu.VMEM((RMAX, 16), jnp.float32),
        pltpu.VMEM((RMAX,), jnp.float32),
        pltpu.VMEM((RMAX,), jnp.float32),
        pltpu.VMEM((RMAX,), jnp.float32),
        pltpu.VMEM((RMAX,), jnp.float32),
        pltpu.VMEM((RMAX,), jnp.int32),
        pltpu.VMEM((NPAD,), jnp.float32),
        pltpu.VMEM((48, 16), jnp.float32),
        pltpu.VMEM((8, 16), jnp.float32),
        pltpu.SMEM((8,), jnp.int32),
        pltpu.SemaphoreType.DMA,
        pltpu.SemaphoreType.DMA,
    ],
)(_sc_body)


def kernel(x, edge_index, batch, W1, b1, p1, W2, b2, p2, W3, b3, p3, W4, b4, p4,
           fc1_W, fc1_b, fc2_W, fc2_b, fc3_W, fc3_b):
    src = edge_index[0]
    dst = edge_index[1]
    xw1 = _matmul(x, W1)
    Wst = jnp.concatenate([W2, W3, W4], axis=0)
    bpst = jnp.stack([
        b1, b2, b3, b4,
        p1 / jnp.linalg.norm(p1), p2 / jnp.linalg.norm(p2),
        p3 / jnp.linalg.norm(p3), p4 / jnp.linalg.norm(p4),
    ])
    x4 = _sc_forward(xw1, src, dst, Wst, bpst)
    out = _head(x4.reshape(1, N0), fc1_W, fc1_b.reshape(1, -1),
                fc2_W, fc2_b.reshape(1, -1), fc3_W, fc3_b.reshape(1, -1))
    return out.reshape(-1)


# speedup vs baseline: 1.9697x; 1.5331x over previous
import functools

import jax
import jax.numpy as jnp
from jax import lax
from jax.experimental import pallas as pl
from jax.experimental.pallas import tpu as pltpu
from jax.experimental.pallas import tpu_sc as plsc

N0 = 1408
E = 90112
T = 16
EPT = E // T
Q = 704
NQ = EPT // Q
STAGE = EPT + Q
NPAD = 1536
DUMPMAX = NPAD - 1

LAYERS = [
    (1408, 704, 88, 1408),
    (704, 352, 48, 768),
    (352, 176, 24, 384),
    (176, 88, 16, 256),
]
RMAX = 96
NEG = -3.0e38


def _mm_body(x_ref, w_ref, o_ref):
    o_ref[...] = jnp.dot(x_ref[...], w_ref[...], preferred_element_type=jnp.float32)


def _matmul(x, w):
    return pl.pallas_call(
        _mm_body,
        out_shape=jax.ShapeDtypeStruct((x.shape[0], w.shape[1]), jnp.float32),
    )(x, w)


def _head_body(x_ref, w1_ref, b1_ref, w2_ref, b2_ref, w3_ref, b3_ref, o_ref):
    h = jnp.dot(x_ref[...], w1_ref[...], preferred_element_type=jnp.float32) + b1_ref[...]
    h = jnp.dot(h, w2_ref[...], preferred_element_type=jnp.float32) + b2_ref[...]
    o_ref[...] = jnp.dot(h, w3_ref[...], preferred_element_type=jnp.float32) + b3_ref[...]


def _head(xf, w1, b1, w2, b2, w3, b3):
    return pl.pallas_call(
        _head_body,
        out_shape=jax.ShapeDtypeStruct((1, 96), jnp.float32),
    )(xf, w1, b1, w2, b2, w3, b3)


def _rsqrt_newton(d):
    i = lax.bitcast_convert_type(d, jnp.int32)
    i = jnp.int32(0x5F3759DF) - (i >> 1)
    y = lax.bitcast_convert_type(i, jnp.float32)
    for _ in range(3):
        y = y * (1.5 - 0.5 * d * y * y)
    return y


def _sc_body(xw1, srcH, dstH, wsH, bpH,
             out,
             sX0, sX1, sXS, sAgg, sDeg, sDinv, sScore, sRank,
             v_msg, v_msg2, v_one, v_src1, v_dst2, v_stD, v_rt,
             v_na, v_nb, v_z, v_zd, v_d, v_sc, v_sc2, v_rk, v_scall,
             v_w, v_bp, m_ref, sem, sem2):
    t = lax.axis_index("s")
    ZR = NPAD // T
    zrow = t * ZR

    def _zfill(r, c):
        v_z[r] = jnp.zeros((16,), jnp.float32)
        return c
    lax.fori_loop(0, RMAX, _zfill, 0)

    def _zdfill(g, c):
        v_zd[pl.ds(g * 16, 16)] = jnp.zeros((16,), jnp.float32)
        return c
    lax.fori_loop(0, RMAX // 16, _zdfill, 0)

    def _ofill(g, c):
        v_one[pl.ds(g * 16, 16)] = jnp.ones((16,), jnp.float32)
        return c
    lax.fori_loop(0, Q // 16, _ofill, 0)
    pltpu.sync_copy(wsH, v_w)
    pltpu.sync_copy(bpH, v_bp)
    pltpu.sync_copy(srcH.at[pl.ds(t * EPT, EPT)], v_src1.at[pl.ds(0, EPT)])
    for q in range(NQ):
        pltpu.sync_copy(dstH.at[pl.ds(t * EPT + q * Q, Q)], v_dst2.at[q])
    pltpu.sync_copy(v_zd, sDeg.at[pl.ds(zrow, ZR)])

    xcur, xnxt = sX0, sX1
    for l, (n_in, k, rpt, NS) in enumerate(LAYERS):
        first = l == 0
        last = l == len(LAYERS) - 1
        ngrp = (rpt + 15) // 16
        row0 = t * rpt
        lanes = lax.iota(jnp.int32, 16)
        GPR = Q // 16
        if first:
            m = EPT
            nq = NQ
        else:
            m = m_ref[0]
            nq = (m + (Q - 1)) // Q

        pltpu.sync_copy(v_z.at[pl.ds(0, ZR)], sAgg.at[pl.ds(zrow, ZR)])
        if first:
            pltpu.sync_copy(xw1.at[pl.ds(t * rpt, rpt)], xcur.at[pl.ds(t * rpt, rpt)])
        else:
            pltpu.sync_copy(xcur.at[pl.ds(row0, rpt)], v_na.at[pl.ds(0, rpt)])

            def _xw_row(r, c):
                xr = v_na[r]
                acc = jnp.zeros((16,), jnp.float32)
                for j in range(16):
                    acc = acc + xr[j] * v_w[(l - 1) * 16 + j]
                v_nb[r] = acc
                return c
            lax.fori_loop(0, rpt, _xw_row, 0)
            pltpu.sync_copy(v_nb.at[pl.ds(0, rpt)], xcur.at[pl.ds(row0, rpt)])
        plsc.subcore_barrier()

        if first:
            for q in range(NQ):
                pltpu.sync_copy(v_one, sDeg.at[v_dst2.at[q]], add=True)
            plsc.subcore_barrier()

        pltpu.sync_copy(sDeg.at[pl.ds(zrow, ZR)], v_d)
        pltpu.sync_copy(xcur.at[pl.ds(zrow, ZR)], v_na)

        def _dinv_grp(g, c):
            base = g * 16
            y = _rsqrt_newton(v_d[pl.ds(base, 16)] + 1.0)
            v_d[pl.ds(base, 16)] = y
            for ri in range(16):
                v_na[base + ri] = y[ri] * v_na[base + ri]
            return c
        lax.fori_loop(0, ZR // 16, _dinv_grp, 0)
        pltpu.sync_copy(v_d, sDinv.at[pl.ds(zrow, ZR)])
        pltpu.sync_copy(v_na, sXS.at[pl.ds(zrow, ZR)])
        plsc.subcore_barrier()

        def _msg_q(q, c):
            pltpu.sync_copy(sXS.at[v_src1.at[pl.ds(q * Q, Q)]], v_msg)
            pltpu.sync_copy(v_msg, sAgg.at[v_dst2.at[q]], add=True)
            return c
        if first:
            bufs = [v_msg, v_msg2]
            sems = [sem, sem2]
            desc = pltpu.async_copy(sXS.at[v_src1.at[pl.ds(0, Q)]], bufs[0], sems[0])
            for q in range(NQ):
                desc.wait()
                if q + 1 < NQ:
                    desc = pltpu.async_copy(
                        sXS.at[v_src1.at[pl.ds((q + 1) * Q, Q)]],
                        bufs[(q + 1) % 2], sems[(q + 1) % 2])
                pltpu.sync_copy(bufs[q % 2], sAgg.at[v_dst2.at[q]], add=True)
        else:
            lax.fori_loop(0, nq, _msg_q, 0)
        plsc.subcore_barrier()

        pltpu.sync_copy(sAgg.at[pl.ds(row0, rpt)], v_na.at[pl.ds(0, rpt)])
        pltpu.sync_copy(sXS.at[pl.ds(row0, rpt)], v_nb.at[pl.ds(0, rpt)])
        pltpu.sync_copy(sDinv.at[pl.ds(row0, rpt)], v_d.at[pl.ds(0, rpt)])
        bvec = v_bp[l]
        pvec = v_bp[4 + l]
        lanes = lax.iota(jnp.int32, 16)

        def _out_grp(g, c):
            base = g * 16
            dv = v_d[pl.ds(base, 16)]
            acc = jnp.full((16,), NEG, jnp.float32)
            for ri in range(16):
                o = dv[ri] * (v_na[base + ri] + v_nb[base + ri]) + bvec
                o = jnp.maximum(o, 0.0)
                v_na[base + ri] = o
                s = o * pvec
                for sh in (8, 4, 2, 1):
                    s = s + s.at[lanes ^ sh].get(mode='promise_in_bounds')
                acc = jnp.where(lanes == ri, s, acc)
            rowg = row0 + base + lanes
            v_sc[pl.ds(base, 16)] = jnp.where(rowg < n_in, acc, NEG)
            return c
        lax.fori_loop(0, ngrp, _out_grp, 0)
        pltpu.sync_copy(v_sc.at[pl.ds(0, rpt)], sScore.at[pl.ds(row0, rpt)])
        plsc.subcore_barrier()

        pltpu.sync_copy(sScore.at[pl.ds(0, NS)], v_scall.at[pl.ds(0, NS)])
        for g in range(ngrp):
            ibase = row0 + g * 16
            siv = v_scall[pl.ds(ibase, 16)]
            iota_i = ibase + lax.iota(jnp.int32, 16)

            nlow = ibase // 16
            nhigh = (ibase + 31) // 16

            def _rank_lo(jg, cnt):
                sjv = v_scall[pl.ds(jg * 16, 16)]
                for jj in range(16):
                    sj = sjv[jj]
                    cnt = cnt + jnp.where((sj > siv) | (sj == siv), 1, 0)
                return cnt

            def _rank_mid(jg, cnt):
                jbase = jg * 16
                sjv = v_scall[pl.ds(jbase, 16)]
                for jj in range(16):
                    sj = sjv[jj]
                    gt = sj > siv
                    eq = (sj == siv) & (jbase + jj < iota_i)
                    cnt = cnt + jnp.where(gt | eq, 1, 0)
                return cnt

            def _rank_hi(jg, cnt):
                sjv = v_scall[pl.ds(jg * 16, 16)]
                for jj in range(16):
                    cnt = cnt + jnp.where(sjv[jj] > siv, 1, 0)
                return cnt
            cnt = lax.fori_loop(0, nlow, _rank_lo,
                                jnp.zeros((16,), jnp.int32))
            cnt = lax.fori_loop(nlow, nhigh, _rank_mid, cnt)
            cnt = lax.fori_loop(nhigh, NS // 16, _rank_hi, cnt)
            if g * 16 + 16 > rpt:
                cnt = jnp.where(lanes < (rpt - g * 16), cnt, DUMPMAX)
            v_rk[pl.ds(g * 16, 16)] = cnt
        for g in range(ngrp, RMAX // 16):
            v_rk[pl.ds(g * 16, 16)] = jnp.full((16,), DUMPMAX, jnp.int32)
        pltpu.sync_copy(v_rk.at[pl.ds(0, rpt)], sRank.at[pl.ds(row0, rpt)])

        for g in range(RMAX // 16):
            sv = v_sc[pl.ds(g * 16, 16)]
            ev = jnp.exp(2.0 * sv)
            v_sc2[pl.ds(g * 16, 16)] = 1.0 - 2.0 / (ev + 1.0)

        def _scale_grp(g, c):
            base = g * 16
            tv = v_sc2[pl.ds(base, 16)]
            for ri in range(16):
                v_nb[base + ri] = v_na[base + ri] * tv[ri]
            return c
        lax.fori_loop(0, ngrp, _scale_grp, 0)
        pltpu.sync_copy(v_nb, xnxt.at[v_rk])
        if not last:
            pltpu.sync_copy(v_zd, sDeg.at[pl.ds(zrow, ZR)])
        plsc.subcore_barrier()

        if not last:
            pltpu.sync_copy(sRank.at[pl.ds(0, NS)], v_rt.at[pl.ds(0, NS)])
            ngq = (m + 15) // 16
            GPR = Q // 16

            def _cmp_g(g, mm):
                off = g * 16
                sv = v_src1[pl.ds(off, 16)]
                dr = g // GPR
                dc = (g % GPR) * 16
                dv = v_dst2[dr, pl.ds(dc, 16)]
                rs = plsc.load_gather(v_rt, [sv])
                rd = plsc.load_gather(v_rt, [dv])
                live = (rs < k) & (rd < k) & (off + lanes < m)
                key = jnp.where(live, 0, 1).astype(jnp.uint32)
                _, rs2 = plsc.sort_key_val(key, rs)
                _, rd2 = plsc.sort_key_val(key, rd)
                v_src1[pl.ds(mm, 16)] = rs2
                v_stD[pl.ds(mm, 16)] = rd2
                pc = plsc.all_reduce_population_count(live)
                return mm + pc[0]
            mm = lax.fori_loop(0, ngq, _cmp_g, jnp.int32(0))
            for g in range(Q // 16):
                v_src1[pl.ds(mm + g * 16, 16)] = jnp.full((16,), k, jnp.int32)
                v_stD[pl.ds(mm + g * 16, 16)] = jnp.full((16,), k, jnp.int32)
            nq2 = (mm + (Q - 1)) // Q

            def _cp_q(q, c):
                for gg in range(Q // 16):
                    v_dst2[q, pl.ds(gg * 16, 16)] = v_stD[pl.ds(q * Q + gg * 16, 16)]
                pltpu.sync_copy(v_one, sDeg.at[v_dst2.at[q]], add=True)
                return c
            lax.fori_loop(0, nq2, _cp_q, 0)
            m_ref[0] = mm
            plsc.subcore_barrier()
        xcur, xnxt = xnxt, xcur

    @pl.when(t == 0)
    def _():
        pltpu.sync_copy(xcur.at[pl.ds(0, 88)], out)


_sc_forward = functools.partial(
    pl.kernel,
    out_type=jax.ShapeDtypeStruct((88, 16), jnp.float32),
    mesh=plsc.VectorSubcoreMesh(core_axis_name="c", subcore_axis_name="s",
                                num_cores=1),
    compiler_params=pltpu.CompilerParams(use_tc_tiling_on_sc=False,
                                         needs_layout_passes=False),
    scratch_types=[
        pltpu.VMEM_SHARED((NPAD, 16), jnp.float32),
        pltpu.VMEM_SHARED((NPAD, 16), jnp.float32),
        pltpu.VMEM_SHARED((NPAD, 16), jnp.float32),
        pltpu.VMEM_SHARED((NPAD, 16), jnp.float32),
        pltpu.VMEM_SHARED((NPAD,), jnp.float32),
        pltpu.VMEM_SHARED((NPAD,), jnp.float32),
        pltpu.VMEM_SHARED((NPAD,), jnp.float32),
        pltpu.VMEM_SHARED((NPAD,), jnp.int32),
        pltpu.VMEM((Q, 16), jnp.float32),
        pltpu.VMEM((Q, 16), jnp.float32),
        pltpu.VMEM((Q,), jnp.float32),
        pltpu.VMEM((STAGE,), jnp.int32),
        pltpu.VMEM((NQ, Q), jnp.int32),
        pltpu.VMEM((STAGE,), jnp.int32),
        pltpu.VMEM((1424,), jnp.int32),
        pltpu.VMEM((RMAX, 16), jnp.float32),
        pltpu.VMEM((RMAX, 16), jnp.float32),
        pltpu.VMEM((RMAX, 16), jnp.float32),
        pltpu.VMEM((RMAX,), jnp.float32),
        pltpu.VMEM((RMAX,), jnp.float32),
        pltpu.VMEM((RMAX,), jnp.float32),
        pltpu.VMEM((RMAX,), jnp.float32),
        pltpu.VMEM((RMAX,), jnp.int32),
        pltpu.VMEM((NPAD,), jnp.float32),
        pltpu.VMEM((48, 16), jnp.float32),
        pltpu.VMEM((8, 16), jnp.float32),
        pltpu.SMEM((8,), jnp.int32),
        pltpu.SemaphoreType.DMA,
        pltpu.SemaphoreType.DMA,
    ],
)(_sc_body)


def kernel(x, edge_index, batch, W1, b1, p1, W2, b2, p2, W3, b3, p3, W4, b4, p4,
           fc1_W, fc1_b, fc2_W, fc2_b, fc3_W, fc3_b):
    src = edge_index[0]
    dst = edge_index[1]
    xw1 = _matmul(x, W1)
    Wst = jnp.concatenate([W2, W3, W4], axis=0)
    bpst = jnp.stack([
        b1, b2, b3, b4,
        p1 / jnp.linalg.norm(p1), p2 / jnp.linalg.norm(p2),
        p3 / jnp.linalg.norm(p3), p4 / jnp.linalg.norm(p4),
    ])
    x4 = _sc_forward(xw1, src, dst, Wst, bpst)
    out = _head(x4.reshape(1, N0), fc1_W, fc1_b.reshape(1, -1),
                fc2_W, fc2_b.reshape(1, -1), fc3_W, fc3_b.reshape(1, -1))
    return out.reshape(-1)
